# Initial kernel scaffold; baseline (speedup 1.0000x reference)
import jax
import jax.numpy as jnp
from jax.experimental import pallas as pl


def kernel(node_attrs, edge_vector, edge_index, W_embed, Wrad, Wmix, Wsq, Wout):
    raise NotImplementedError("write your pallas kernel here")


# speedup vs baseline: 26.7876x; 26.7876x over previous
import functools

import jax
import jax.numpy as jnp
from jax import lax
from jax.experimental import pallas as pl
from jax.experimental.pallas import tpu as pltpu
from jax.experimental.pallas import tpu_sc as plsc

N_NODES = 10000
N_EDGES = 160000
C = 64
NUM_BASIS = 8
CUTOFF = 6.0
AVG_NEIGH = 16.0

NC = 2
NS = 16
CH = 128
NCHUNK = N_EDGES // CH
HALFCHUNK = NCHUNK // 2
ROWS_PER_TILE = N_NODES // NS

SYM_PAIRS = ((0, 0), (0, 1), (0, 2), (1, 1), (1, 2), (2, 2))
SYM_W = (1.0, 2.0, 2.0, 1.0, 2.0, 1.0)


def _gather_body(table, idx_hbm, out, idx_v, rows_v, sem):
    wid = lax.axis_index("s") * NC + lax.axis_index("c")
    nw = NC * NS
    nloop = (NCHUNK + nw - 1) // nw

    def step(j, carry):
        cid = wid + nw * j

        @pl.when(cid < NCHUNK)
        def _():
            base = pl.multiple_of(cid * CH, CH)
            pltpu.sync_copy(idx_hbm.at[pl.ds(base, CH)], idx_v)
            pltpu.async_copy(table.at[idx_v], rows_v, sem).wait()
            pltpu.sync_copy(rows_v, out.at[pl.ds(base, CH)])

        return carry

    lax.fori_loop(0, nloop, step, 0)


def _sc_gather(table, idx):
    d = table.shape[1]
    mesh = plsc.VectorSubcoreMesh(core_axis_name="c", subcore_axis_name="s")
    f = pl.kernel(
        _gather_body,
        out_type=jax.ShapeDtypeStruct((N_EDGES, d), jnp.float32),
        mesh=mesh,
        scratch_types=[
            pltpu.VMEM((CH,), jnp.int32),
            pltpu.VMEM((CH, d), jnp.float32),
            pltpu.SemaphoreType.DMA,
        ],
    )
    return f(table, idx)


_SCAT_PLAN = (
    ((0, 0, 0, NCHUNK), (1, 1, 0, NCHUNK), (4, 4, 0, HALFCHUNK)),
    ((2, 2, 0, NCHUNK), (3, 3, 0, NCHUNK), (4, 5, HALFCHUNK, NCHUNK)),
)


_TROWS = 632
_TROWS_LAST = N_NODES - 15 * _TROWS


def _scatter_body(msg_hbm, dst_hbm, zeros_hbm, out_hbm, idx_v, mbuf, acc):
    s = lax.axis_index("c")
    t = lax.axis_index("s")

    def each_tile_rows(fn):
        @pl.when(t < 15)
        def _():
            fn(pl.multiple_of(t * _TROWS, 8), _TROWS)

        @pl.when(t == 15)
        def _():
            fn(15 * _TROWS, _TROWS_LAST)

    for sv in range(2):

        @pl.when(s == sv)
        def _(sv=sv):
            for (slot, slab, lo, hi) in _SCAT_PLAN[sv]:
                each_tile_rows(lambda r0, nr: pltpu.sync_copy(
                    zeros_hbm.at[pl.ds(r0, nr)], acc.at[pl.ds(r0, nr)]))
                plsc.subcore_barrier()
                nloop = (hi - lo + NS - 1) // NS

                def step(j, carry, slot=slot, lo=lo, hi=hi):
                    cid = lo + t + NS * j

                    @pl.when(cid < hi)
                    def _():
                        base = pl.multiple_of(cid * CH, CH)
                        pltpu.sync_copy(dst_hbm.at[pl.ds(base, CH)], idx_v)
                        pltpu.sync_copy(
                            msg_hbm.at[pl.ds(base, CH),
                                       pl.ds(slot * 128, 128)], mbuf)
                        pltpu.sync_copy(mbuf, acc.at[idx_v], add=True)

                    return carry

                lax.fori_loop(0, nloop, step, 0)
                plsc.subcore_barrier()
                each_tile_rows(lambda r0, nr, slab=slab: pltpu.sync_copy(
                    acc.at[pl.ds(r0, nr)],
                    out_hbm.at[slab, pl.ds(r0, nr)]))
                plsc.subcore_barrier()


def _sc_scatter(msg, dst, zeros_acc):
    mesh = plsc.VectorSubcoreMesh(core_axis_name="c", subcore_axis_name="s")
    f = pl.kernel(
        _scatter_body,
        out_type=jax.ShapeDtypeStruct((6, N_NODES, 128), jnp.float32),
        mesh=mesh,
        scratch_types=[
            pltpu.VMEM((CH,), jnp.int32),
            pltpu.VMEM((CH, 128), jnp.float32),
            pltpu.VMEM_SHARED((N_NODES, 128), jnp.float32),
        ],
    )
    return f(msg, dst, zeros_acc)


NBK = 2000


def _embed_body(at_ref, we_ref, out_ref):
    h = at_ref[...] @ we_ref[...]
    out_ref[...] = jnp.concatenate([h, jnp.zeros_like(h)], axis=1)


def _embed(attrs16, we16):
    grid = N_NODES // NBK
    return pl.pallas_call(
        _embed_body,
        grid=(grid,),
        in_specs=[
            pl.BlockSpec((NBK, 16), lambda i: (i, 0)),
            pl.BlockSpec((16, C), lambda i: (0, 0)),
        ],
        out_specs=pl.BlockSpec((NBK, 2 * C), lambda i: (i, 0)),
        out_shape=jax.ShapeDtypeStruct((N_NODES, 2 * C), jnp.float32),
    )(attrs16, we16)


def _edge_geom(ev):
    r2 = jnp.sum(ev * ev, axis=1, keepdims=True)
    r = jnp.sqrt(r2)
    rhat = ev / r
    u = r / CUTOFF
    u2 = u * u
    u4 = u2 * u2
    u5 = u4 * u
    u6 = u5 * u
    u7 = u6 * u
    fc = jnp.where(u < 1.0, 1.0 - 21.0 * u5 + 35.0 * u6 - 15.0 * u7, 0.0)
    n = (jnp.arange(1, NUM_BASIS + 1)[None, :]).astype(jnp.float32)
    bessel = jnp.sqrt(2.0 / CUTOFF) * jnp.sin(n * jnp.pi * u) / r
    ef = bessel * fc
    return rhat, ef


def _write_msg(msg_ref, rhat, ef, g, wrad_ref):
    r0 = ef @ wrad_ref[0]
    r1 = ef @ wrad_ref[1]
    r2 = ef @ wrad_ref[2]
    p0 = r0 * g
    p1 = r1 * g
    p2 = r2 * g
    comps = [p0]
    for d in range(3):
        comps.append(rhat[:, d:d + 1] * p1)
    for (i, j) in SYM_PAIRS:
        comps.append((rhat[:, i:i + 1] * rhat[:, j:j + 1]) * p2)
    msg_ref[...] = jnp.concatenate(comps, axis=1)


def _edge_body_l0(ev_ref, gat_ref, wrad_ref, msg_ref):
    rhat, ef = _edge_geom(ev_ref[...])
    g = gat_ref[:, 0:C]
    _write_msg(msg_ref, rhat, ef, g, wrad_ref)


def _edge_body_l1(ev_ref, gat_ref, wrad_ref, msg_ref):
    rhat, ef = _edge_geom(ev_ref[...])
    g = gat_ref[:, 0:C]
    for d in range(3):
        g = g + rhat[:, d:d + 1] * gat_ref[:, (1 + d) * C:(2 + d) * C]
    _write_msg(msg_ref, rhat, ef, g, wrad_ref)


EB = 1280


def _edge_msg(body, gat, edge_vector, wrad_l):
    grid = N_EDGES // EB
    d = gat.shape[1]
    return pl.pallas_call(
        body,
        grid=(grid,),
        in_specs=[
            pl.BlockSpec((EB, 3), lambda i: (i, 0)),
            pl.BlockSpec((EB, d), lambda i: (i, 0)),
            pl.BlockSpec((3, NUM_BASIS, C), lambda i: (0, 0, 0)),
        ],
        out_specs=pl.BlockSpec((EB, 10 * C), lambda i: (i, 0)),
        out_shape=jax.ShapeDtypeStruct((N_EDGES, 10 * C), jnp.float32),
    )(edge_vector, gat, wrad_l)


def _agg_comp(a, k):
    lo, hi = (k % 2) * C, (k % 2) * C + C
    if k < 8:
        return a[k // 2, :, lo:hi]
    return a[4, :, lo:hi] + a[5, :, lo:hi]


def _node_mix_inv(agg_ref, wmix_ref, wsq_ref):
    a = agg_ref[...] * (1.0 / AVG_NEIGH)
    a0 = _agg_comp(a, 0) @ wmix_ref[0]
    a1 = [_agg_comp(a, 1 + d) @ wmix_ref[1] for d in range(3)]
    a2 = [_agg_comp(a, 4 + k) @ wmix_ref[2] for k in range(6)]
    inv = (a0 * a0) @ wsq_ref[0]
    s1 = a1[0] * a1[0] + a1[1] * a1[1] + a1[2] * a1[2]
    inv = inv + s1 @ wsq_ref[1]
    s2 = SYM_W[0] * (a2[0] * a2[0])
    for k in range(1, 6):
        s2 = s2 + SYM_W[k] * (a2[k] * a2[k])
    inv = inv + s2 @ wsq_ref[2]
    return a0, a1, inv


def _node_body_l0(agg_ref, wmix_ref, wsq_ref, wout_ref, pk_ref):
    a0, a1, inv = _node_mix_inv(agg_ref, wmix_ref, wsq_ref)
    h0 = a0 @ wout_ref[0] + inv
    h1 = [a1[d] @ wout_ref[1] for d in range(3)]
    pk_ref[...] = jnp.concatenate([h0] + h1, axis=1)


def _node_body_l1(agg_ref, wmix_ref, wsq_ref, wout_ref, out_ref):
    a0, _, inv = _node_mix_inv(agg_ref, wmix_ref, wsq_ref)
    out_ref[...] = a0 @ wout_ref[0] + inv


def _node(body, out_minor, agg, wmix, wsq, wout):
    grid = N_NODES // NBK
    return pl.pallas_call(
        body,
        grid=(grid,),
        in_specs=[
            pl.BlockSpec((6, NBK, 2 * C), lambda i: (0, i, 0)),
            pl.BlockSpec((3, C, C), lambda i: (0, 0, 0)),
            pl.BlockSpec((3, C, C), lambda i: (0, 0, 0)),
            pl.BlockSpec((2, C, C), lambda i: (0, 0, 0)),
        ],
        out_specs=pl.BlockSpec((NBK, out_minor), lambda i: (i, 0)),
        out_shape=jax.ShapeDtypeStruct((N_NODES, out_minor), jnp.float32),
    )(agg, wmix, wsq, wout)


def kernel(node_attrs, edge_vector, edge_index, W_embed, Wrad, Wmix, Wsq, Wout):
    src = edge_index[0].astype(jnp.int32)
    dst = edge_index[1].astype(jnp.int32)
    n_sp = node_attrs.shape[1]
    attrs16 = jnp.pad(node_attrs, ((0, 0), (0, 16 - n_sp)))
    we16 = jnp.pad(W_embed, ((0, 16 - n_sp), (0, 0)))
    zeros_acc = jnp.zeros((N_NODES, 128), jnp.float32)

    table0 = _embed(attrs16, we16)
    gat0 = _sc_gather(table0, src)
    msg0 = _edge_msg(_edge_body_l0, gat0, edge_vector, Wrad[0])
    agg0 = _sc_scatter(msg0, dst, zeros_acc)
    packed = _node(_node_body_l0, 4 * C, agg0, Wmix[0], Wsq[0], Wout[0])

    gat1 = _sc_gather(packed, src)
    msg1 = _edge_msg(_edge_body_l1, gat1, edge_vector, Wrad[1])
    agg1 = _sc_scatter(msg1, dst, zeros_acc)
    return _node(_node_body_l1, C, agg1, Wmix[1], Wsq[1], Wout[1])


# speedup vs baseline: 29.6840x; 1.1081x over previous
import functools

import jax
import jax.numpy as jnp
from jax import lax
from jax.experimental import pallas as pl
from jax.experimental.pallas import tpu as pltpu
from jax.experimental.pallas import tpu_sc as plsc

N_NODES = 10000
N_EDGES = 160000
C = 64
NUM_BASIS = 8
CUTOFF = 6.0
AVG_NEIGH = 16.0

NC = 2
NS = 16
CH = 128
NCHUNK = N_EDGES // CH
NCHUNK_PAD = 1280
PER_G = NCHUNK_PAD // (NC * NS)
SPLITCHUNK = 624
LAG = 16

SYM_PAIRS = ((0, 0), (0, 1), (0, 2), (1, 1), (1, 2), (2, 2))
SYM_W = (1.0, 2.0, 2.0, 1.0, 2.0, 1.0)


def _gather_body_simple(table, idx_hbm, out, idx_v, rows_v, sem):
    wid = lax.axis_index("s") * NC + lax.axis_index("c")
    nw = NC * NS
    nloop = (NCHUNK + nw - 1) // nw

    def step(j, carry):
        cid = wid + nw * j

        @pl.when(cid < NCHUNK)
        def _():
            base = pl.multiple_of(cid * CH, CH)
            pltpu.sync_copy(idx_hbm.at[pl.ds(base, CH)], idx_v)
            pltpu.async_copy(table.at[idx_v], rows_v, sem).wait()
            pltpu.sync_copy(rows_v, out.at[pl.ds(base, CH)])

        return carry

    lax.fori_loop(0, nloop, step, 0)


def _gather_body(nbuf, table, idx2, out, ibuf,
                 r0, r1, r2, r3, sg0, sg1, sg2, sg3, so0, so1, so2, so3):
    rows = (r0, r1, r2, r3)
    sgs = (sg0, sg1, sg2, sg3)
    sos = (so0, so1, so2, so3)
    wid = lax.axis_index("s") * NC + lax.axis_index("c")
    c0 = pl.multiple_of(wid * PER_G, 8)
    pltpu.sync_copy(idx2.at[pl.ds(c0, PER_G)], ibuf)
    nouter = (PER_G + nbuf - 1) // nbuf

    def outer(g, carry):
        for b in range(nbuf):
            j = g * nbuf + b
            cid = c0 + j

            @pl.when((j < PER_G) & (cid < NCHUNK))
            def _(b=b, j=j, g=g):
                @pl.when(g > 0)
                def _():
                    pltpu.make_async_copy(
                        table.at[pl.ds(0, CH)], rows[b], sos[b]).wait()

                pltpu.async_copy(table.at[ibuf.at[j]], rows[b], sgs[b])

        for b in range(nbuf):
            j = g * nbuf + b
            cid = c0 + j

            @pl.when((j < PER_G) & (cid < NCHUNK))
            def _(b=b, cid=cid):
                pltpu.make_async_copy(
                    table.at[pl.ds(0, CH)], rows[b], sgs[b]).wait()
                base = pl.multiple_of(cid * CH, CH)
                pltpu.async_copy(rows[b], out.at[pl.ds(base, CH)], sos[b])

        return carry

    lax.fori_loop(0, nouter, outer, 0)
    for b in range(nbuf):
        @pl.when(c0 + b < NCHUNK)
        def _(b=b):
            pltpu.make_async_copy(
                table.at[pl.ds(0, CH)], rows[b], sos[b]).wait()


def _sc_gather(table, idx_flat):
    d = table.shape[1]
    mesh = plsc.VectorSubcoreMesh(core_axis_name="c", subcore_axis_name="s")
    f = pl.kernel(
        _gather_body_simple,
        out_type=jax.ShapeDtypeStruct((N_EDGES, d), jnp.float32),
        mesh=mesh,
        scratch_types=[
            pltpu.VMEM((CH,), jnp.int32),
            pltpu.VMEM((CH, d), jnp.float32),
            pltpu.SemaphoreType.DMA,
        ],
    )
    return f(table, idx_flat)


_SCAT_PLAN = (
    ((0, 0, 0, NCHUNK, 80), (1, 1, 0, NCHUNK, 80),
     (4, 4, 0, SPLITCHUNK, 40)),
    ((2, 2, 0, NCHUNK, 80), (3, 3, 0, NCHUNK, 80),
     (4, 5, SPLITCHUNK, NCHUNK, 40)),
)


_TROWS = 632
_TROWS_LAST = N_NODES - 15 * _TROWS


def _scatter_body(msg_hbm, dst_hbm, zeros_hbm, out_hbm,
                  i0, i1, m0, m1, acc, sl0, sl1, ss0, ss1):
    s = lax.axis_index("c")
    t = lax.axis_index("s")
    ib = (i0, i1)
    mb = (m0, m1)
    sls = (sl0, sl1)
    sss = (ss0, ss1)

    def each_tile_rows(fn):
        @pl.when(t < 15)
        def _():
            fn(pl.multiple_of(t * _TROWS, 8), _TROWS)

        @pl.when(t == 15)
        def _():
            fn(15 * _TROWS, _TROWS_LAST)

    def drain_load(b):
        pltpu.make_async_copy(
            dst_hbm.at[pl.ds(0, CH)], ib[b], sls[b]).wait()
        pltpu.make_async_copy(
            msg_hbm.at[pl.ds(0, CH), pl.ds(0, 128)], mb[b], sls[b]).wait()

    def drain_scat(b):
        pltpu.make_async_copy(mb[b], acc.at[ib[b]], sss[b]).wait()

    for sv in range(2):

        @pl.when(s == sv)
        def _(sv=sv):
            for (slot, slab, lo, hi, per) in _SCAT_PLAN[sv]:
                each_tile_rows(lambda r0, nr: pltpu.sync_copy(
                    zeros_hbm.at[pl.ds(r0, nr)], acc.at[pl.ds(r0, nr)]))
                plsc.subcore_barrier()
                c0 = pl.multiple_of(lo + t * per, 8)

                def body(g, carry, slot=slot, c0=c0, hi=hi):
                    for b in range(2):
                        j = 2 * g + b
                        cid = c0 + j

                        @pl.when(cid < hi)
                        def _(b=b, cid=cid, g=g):
                            @pl.when(g > 0)
                            def _():
                                drain_scat(b)

                            base = pl.multiple_of(cid * CH, CH)
                            pltpu.async_copy(
                                dst_hbm.at[pl.ds(base, CH)], ib[b], sls[b])
                            pltpu.async_copy(
                                msg_hbm.at[pl.ds(base, CH),
                                           pl.ds(slot * 128, 128)],
                                mb[b], sls[b])

                    for b in range(2):
                        j = 2 * g + b
                        cid = c0 + j

                        @pl.when(cid < hi)
                        def _(b=b):
                            drain_load(b)
                            pltpu.async_copy(mb[b], acc.at[ib[b]],
                                             sss[b], add=True)

                    return carry

                lax.fori_loop(0, per // 2, body, 0)
                for b in range(2):
                    @pl.when(c0 + b < hi)
                    def _(b=b):
                        drain_scat(b)

                plsc.subcore_barrier()
                each_tile_rows(lambda r0, nr, slab=slab: pltpu.sync_copy(
                    acc.at[pl.ds(r0, nr)],
                    out_hbm.at[slab, pl.ds(r0, nr)]))
                plsc.subcore_barrier()


def _sc_scatter(msg, dst_flat, zeros_acc):
    mesh = plsc.VectorSubcoreMesh(core_axis_name="c", subcore_axis_name="s")
    f = pl.kernel(
        _scatter_body,
        out_type=jax.ShapeDtypeStruct((6, N_NODES, 128), jnp.float32),
        mesh=mesh,
        scratch_types=(
            [pltpu.VMEM((CH,), jnp.int32)] * 2
            + [pltpu.VMEM((CH, 128), jnp.float32)] * 2
            + [pltpu.VMEM_SHARED((N_NODES, 128), jnp.float32)]
            + [pltpu.SemaphoreType.DMA] * 4
        ),
    )
    return f(msg, dst_flat, zeros_acc)


NBK = 2000


def _embed_body(at_ref, we_ref, out_ref):
    h = at_ref[...] @ we_ref[...]
    out_ref[...] = jnp.concatenate([h, jnp.zeros_like(h)], axis=1)


def _embed(attrs16, we16):
    grid = N_NODES // NBK
    return pl.pallas_call(
        _embed_body,
        grid=(grid,),
        in_specs=[
            pl.BlockSpec((NBK, 16), lambda i: (i, 0)),
            pl.BlockSpec((16, C), lambda i: (0, 0)),
        ],
        out_specs=pl.BlockSpec((NBK, 2 * C), lambda i: (i, 0)),
        out_shape=jax.ShapeDtypeStruct((N_NODES, 2 * C), jnp.float32),
    )(attrs16, we16)


def _edge_geom(evt):
    ev = evt.T
    r2 = jnp.sum(ev * ev, axis=1, keepdims=True)
    r = jnp.sqrt(r2)
    rhat = ev / r
    u = r / CUTOFF
    u2 = u * u
    u4 = u2 * u2
    u5 = u4 * u
    u6 = u5 * u
    u7 = u6 * u
    fc = jnp.where(u < 1.0, 1.0 - 21.0 * u5 + 35.0 * u6 - 15.0 * u7, 0.0)
    n = (jnp.arange(1, NUM_BASIS + 1)[None, :]).astype(jnp.float32)
    bessel = jnp.sqrt(2.0 / CUTOFF) * jnp.sin(n * jnp.pi * u) / r
    ef = bessel * fc
    return rhat, ef


def _write_msg(msg_ref, rhat, ef, g, wrad_ref):
    r0 = ef @ wrad_ref[0]
    r1 = ef @ wrad_ref[1]
    r2 = ef @ wrad_ref[2]
    p0 = r0 * g
    p1 = r1 * g
    p2 = r2 * g
    comps = [p0]
    for d in range(3):
        comps.append(rhat[:, d:d + 1] * p1)
    for (i, j) in SYM_PAIRS:
        comps.append((rhat[:, i:i + 1] * rhat[:, j:j + 1]) * p2)
    msg_ref[...] = jnp.concatenate(comps, axis=1)


def _edge_body_l0(ev_ref, gat_ref, wrad_ref, msg_ref):
    rhat, ef = _edge_geom(ev_ref[...])
    g = gat_ref[:, 0:C]
    _write_msg(msg_ref, rhat, ef, g, wrad_ref)


def _edge_body_l1(ev_ref, gat_ref, wrad_ref, msg_ref):
    rhat, ef = _edge_geom(ev_ref[...])
    g = gat_ref[:, 0:C]
    for d in range(3):
        g = g + rhat[:, d:d + 1] * gat_ref[:, (1 + d) * C:(2 + d) * C]
    _write_msg(msg_ref, rhat, ef, g, wrad_ref)


EB = 1280


def _edge_msg(body, gat, edge_vector, wrad_l):
    grid = N_EDGES // EB
    d = gat.shape[1]
    return pl.pallas_call(
        body,
        grid=(grid,),
        in_specs=[
            pl.BlockSpec((3, EB), lambda i: (0, i)),
            pl.BlockSpec((EB, d), lambda i: (i, 0)),
            pl.BlockSpec((3, NUM_BASIS, C), lambda i: (0, 0, 0)),
        ],
        out_specs=pl.BlockSpec((EB, 10 * C), lambda i: (i, 0)),
        out_shape=jax.ShapeDtypeStruct((N_EDGES, 10 * C), jnp.float32),
    )(edge_vector, gat, wrad_l)


def _agg_comp(a, k):
    lo, hi = (k % 2) * C, (k % 2) * C + C
    if k < 8:
        return a[k // 2, :, lo:hi]
    return a[4, :, lo:hi] + a[5, :, lo:hi]


def _node_mix_inv(agg_ref, wmix_ref, wsq_ref):
    a = agg_ref[...] * (1.0 / AVG_NEIGH)
    a0 = _agg_comp(a, 0) @ wmix_ref[0]
    a1 = [_agg_comp(a, 1 + d) @ wmix_ref[1] for d in range(3)]
    a2 = [_agg_comp(a, 4 + k) @ wmix_ref[2] for k in range(6)]
    inv = (a0 * a0) @ wsq_ref[0]
    s1 = a1[0] * a1[0] + a1[1] * a1[1] + a1[2] * a1[2]
    inv = inv + s1 @ wsq_ref[1]
    s2 = SYM_W[0] * (a2[0] * a2[0])
    for k in range(1, 6):
        s2 = s2 + SYM_W[k] * (a2[k] * a2[k])
    inv = inv + s2 @ wsq_ref[2]
    return a0, a1, inv


def _node_body_l0(agg_ref, wmix_ref, wsq_ref, wout_ref, pk_ref):
    a0, a1, inv = _node_mix_inv(agg_ref, wmix_ref, wsq_ref)
    h0 = a0 @ wout_ref[0] + inv
    h1 = [a1[d] @ wout_ref[1] for d in range(3)]
    pk_ref[...] = jnp.concatenate([h0] + h1, axis=1)


def _node_body_l1(agg_ref, wmix_ref, wsq_ref, wout_ref, out_ref):
    a0, _, inv = _node_mix_inv(agg_ref, wmix_ref, wsq_ref)
    out_ref[...] = a0 @ wout_ref[0] + inv


def _node(body, out_minor, agg, wmix, wsq, wout):
    grid = N_NODES // NBK
    return pl.pallas_call(
        body,
        grid=(grid,),
        in_specs=[
            pl.BlockSpec((6, NBK, 2 * C), lambda i: (0, i, 0)),
            pl.BlockSpec((3, C, C), lambda i: (0, 0, 0)),
            pl.BlockSpec((3, C, C), lambda i: (0, 0, 0)),
            pl.BlockSpec((2, C, C), lambda i: (0, 0, 0)),
        ],
        out_specs=pl.BlockSpec((NBK, out_minor), lambda i: (i, 0)),
        out_shape=jax.ShapeDtypeStruct((N_NODES, out_minor), jnp.float32),
    )(agg, wmix, wsq, wout)


def kernel(node_attrs, edge_vector, edge_index, W_embed, Wrad, Wmix, Wsq, Wout):
    src_i = edge_index[0].astype(jnp.int32)
    dst = edge_index[1].astype(jnp.int32)
    evT = edge_vector.T
    n_sp = node_attrs.shape[1]
    attrs16 = jnp.pad(node_attrs, ((0, 0), (0, 16 - n_sp)))
    we16 = jnp.pad(W_embed, ((0, 16 - n_sp), (0, 0)))
    zeros_acc = jnp.zeros((N_NODES, 128), jnp.float32)

    table0 = _embed(attrs16, we16)
    gat0 = _sc_gather(table0, src_i)
    msg0 = _edge_msg(_edge_body_l0, gat0, evT, Wrad[0])
    agg0 = _sc_scatter(msg0, dst, zeros_acc)
    packed = _node(_node_body_l0, 4 * C, agg0, Wmix[0], Wsq[0], Wout[0])

    gat1 = _sc_gather(packed, src_i)
    msg1 = _edge_msg(_edge_body_l1, gat1, evT, Wrad[1])
    agg1 = _sc_scatter(msg1, dst, zeros_acc)
    return _node(_node_body_l1, C, agg1, Wmix[1], Wsq[1], Wout[1])


# speedup vs baseline: 30.6237x; 1.0317x over previous
import functools

import jax
import jax.numpy as jnp
from jax import lax
from jax.experimental import pallas as pl
from jax.experimental.pallas import tpu as pltpu
from jax.experimental.pallas import tpu_sc as plsc

N_NODES = 10000
N_EDGES = 160000
C = 64
NUM_BASIS = 8
CUTOFF = 6.0
AVG_NEIGH = 16.0

NC = 2
NS = 16
CH = 128
NCHUNK = N_EDGES // CH
NCHUNK_PAD = 1280
PER_G = NCHUNK_PAD // (NC * NS)
SPLITCHUNK = 624
LAG = 16

SYM_PAIRS = ((0, 0), (0, 1), (0, 2), (1, 1), (1, 2), (2, 2))
SYM_W = (1.0, 2.0, 2.0, 1.0, 2.0, 1.0)


def _gather_body_simple(table, idx_hbm, out, idx_v, rows_v, sem):
    wid = lax.axis_index("s") * NC + lax.axis_index("c")
    nw = NC * NS
    nloop = (NCHUNK + nw - 1) // nw

    def step(j, carry):
        cid = wid + nw * j

        @pl.when(cid < NCHUNK)
        def _():
            base = pl.multiple_of(cid * CH, CH)
            pltpu.sync_copy(idx_hbm.at[pl.ds(base, CH)], idx_v)
            pltpu.async_copy(table.at[idx_v], rows_v, sem).wait()
            pltpu.sync_copy(rows_v, out.at[pl.ds(base, CH)])

        return carry

    lax.fori_loop(0, nloop, step, 0)


def _gather_body(nbuf, table, idx2, out, ibuf,
                 r0, r1, r2, r3, sg0, sg1, sg2, sg3, so0, so1, so2, so3):
    rows = (r0, r1, r2, r3)
    sgs = (sg0, sg1, sg2, sg3)
    sos = (so0, so1, so2, so3)
    wid = lax.axis_index("s") * NC + lax.axis_index("c")
    c0 = pl.multiple_of(wid * PER_G, 8)
    pltpu.sync_copy(idx2.at[pl.ds(c0, PER_G)], ibuf)
    nouter = (PER_G + nbuf - 1) // nbuf

    def outer(g, carry):
        for b in range(nbuf):
            j = g * nbuf + b
            cid = c0 + j

            @pl.when((j < PER_G) & (cid < NCHUNK))
            def _(b=b, j=j, g=g):
                @pl.when(g > 0)
                def _():
                    pltpu.make_async_copy(
                        table.at[pl.ds(0, CH)], rows[b], sos[b]).wait()

                pltpu.async_copy(table.at[ibuf.at[j]], rows[b], sgs[b])

        for b in range(nbuf):
            j = g * nbuf + b
            cid = c0 + j

            @pl.when((j < PER_G) & (cid < NCHUNK))
            def _(b=b, j=j, cid=cid):
                pltpu.make_async_copy(
                    table.at[ibuf.at[j]], rows[b], sgs[b]).wait()
                base = pl.multiple_of(cid * CH, CH)
                pltpu.async_copy(rows[b], out.at[pl.ds(base, CH)], sos[b])

        return carry

    lax.fori_loop(0, nouter, outer, 0)
    for b in range(nbuf):
        @pl.when(c0 + b < NCHUNK)
        def _(b=b):
            pltpu.make_async_copy(
                table.at[pl.ds(0, CH)], rows[b], sos[b]).wait()


def _sc_gather(table, idx2):
    d = table.shape[1]
    nbuf = 4 if d <= 128 else 3
    mesh = plsc.VectorSubcoreMesh(core_axis_name="c", subcore_axis_name="s")
    f = pl.kernel(
        functools.partial(_gather_body, nbuf),
        out_type=jax.ShapeDtypeStruct((N_EDGES, d), jnp.float32),
        mesh=mesh,
        scratch_types=(
            [pltpu.VMEM((PER_G, CH), jnp.int32)]
            + [pltpu.VMEM((CH, d), jnp.float32)] * nbuf
            + [pltpu.VMEM((8, 128), jnp.float32)] * (4 - nbuf)
            + [pltpu.SemaphoreType.DMA] * 8
        ),
    )
    return f(table, idx2)


_SCAT_PLAN = (
    ((0, 0, 0, NCHUNK, 80), (1, 1, 0, NCHUNK, 80),
     (4, 4, 0, SPLITCHUNK, 40)),
    ((2, 2, 0, NCHUNK, 80), (3, 3, 0, NCHUNK, 80),
     (4, 5, SPLITCHUNK, NCHUNK, 40)),
)


_TROWS = 632
_TROWS_LAST = N_NODES - 15 * _TROWS


def _scatter_body(msg_hbm, dst_hbm, zeros_hbm, out_hbm,
                  i0, i1, m0, m1, acc, sl0, sl1, ss0, ss1):
    s = lax.axis_index("c")
    t = lax.axis_index("s")
    ib = (i0, i1)
    mb = (m0, m1)
    sls = (sl0, sl1)
    sss = (ss0, ss1)

    def each_tile_rows(fn):
        @pl.when(t < 15)
        def _():
            fn(pl.multiple_of(t * _TROWS, 8), _TROWS)

        @pl.when(t == 15)
        def _():
            fn(15 * _TROWS, _TROWS_LAST)

    def drain_load(b):
        pltpu.make_async_copy(
            dst_hbm.at[pl.ds(0, CH)], ib[b], sls[b]).wait()
        pltpu.make_async_copy(
            msg_hbm.at[pl.ds(0, CH), pl.ds(0, 128)], mb[b], sls[b]).wait()

    def drain_scat(b):
        pltpu.make_async_copy(mb[b], acc.at[ib[b]], sss[b]).wait()

    for sv in range(2):

        @pl.when(s == sv)
        def _(sv=sv):
            for (slot, slab, lo, hi, per) in _SCAT_PLAN[sv]:
                each_tile_rows(lambda r0, nr: pltpu.sync_copy(
                    zeros_hbm.at[pl.ds(r0, nr)], acc.at[pl.ds(r0, nr)]))
                plsc.subcore_barrier()
                c0 = pl.multiple_of(lo + t * per, 8)

                def body(g, carry, slot=slot, c0=c0, hi=hi):
                    for b in range(2):
                        j = 2 * g + b
                        cid = c0 + j

                        @pl.when(cid < hi)
                        def _(b=b, cid=cid, g=g):
                            @pl.when(g > 0)
                            def _():
                                drain_scat(b)

                            base = pl.multiple_of(cid * CH, CH)
                            pltpu.async_copy(
                                dst_hbm.at[pl.ds(base, CH)], ib[b], sls[b])
                            pltpu.async_copy(
                                msg_hbm.at[pl.ds(base, CH),
                                           pl.ds(slot * 128, 128)],
                                mb[b], sls[b])

                    for b in range(2):
                        j = 2 * g + b
                        cid = c0 + j

                        @pl.when(cid < hi)
                        def _(b=b):
                            drain_load(b)
                            pltpu.async_copy(mb[b], acc.at[ib[b]],
                                             sss[b], add=True)

                    return carry

                lax.fori_loop(0, per // 2, body, 0)
                for b in range(2):
                    @pl.when(c0 + b < hi)
                    def _(b=b):
                        drain_scat(b)

                plsc.subcore_barrier()
                each_tile_rows(lambda r0, nr, slab=slab: pltpu.sync_copy(
                    acc.at[pl.ds(r0, nr)],
                    out_hbm.at[slab, pl.ds(r0, nr)]))
                plsc.subcore_barrier()


def _sc_scatter(msg, dst_flat, zeros_acc):
    mesh = plsc.VectorSubcoreMesh(core_axis_name="c", subcore_axis_name="s")
    f = pl.kernel(
        _scatter_body,
        out_type=jax.ShapeDtypeStruct((6, N_NODES, 128), jnp.float32),
        mesh=mesh,
        scratch_types=(
            [pltpu.VMEM((CH,), jnp.int32)] * 2
            + [pltpu.VMEM((CH, 128), jnp.float32)] * 2
            + [pltpu.VMEM_SHARED((N_NODES, 128), jnp.float32)]
            + [pltpu.SemaphoreType.DMA] * 4
        ),
    )
    return f(msg, dst_flat, zeros_acc)


NBK = 2000


def _embed_body(at_ref, we_ref, out_ref):
    h = at_ref[...] @ we_ref[...]
    out_ref[...] = jnp.concatenate([h, jnp.zeros_like(h)], axis=1)


def _embed(attrs16, we16):
    grid = N_NODES // NBK
    return pl.pallas_call(
        _embed_body,
        grid=(grid,),
        in_specs=[
            pl.BlockSpec((NBK, 16), lambda i: (i, 0)),
            pl.BlockSpec((16, C), lambda i: (0, 0)),
        ],
        out_specs=pl.BlockSpec((NBK, 2 * C), lambda i: (i, 0)),
        out_shape=jax.ShapeDtypeStruct((N_NODES, 2 * C), jnp.float32),
    )(attrs16, we16)


def _edge_geom(evt):
    ev = evt.T
    r2 = jnp.sum(ev * ev, axis=1, keepdims=True)
    r = jnp.sqrt(r2)
    rhat = ev / r
    u = r / CUTOFF
    u2 = u * u
    u4 = u2 * u2
    u5 = u4 * u
    u6 = u5 * u
    u7 = u6 * u
    fc = jnp.where(u < 1.0, 1.0 - 21.0 * u5 + 35.0 * u6 - 15.0 * u7, 0.0)
    n = (jnp.arange(1, NUM_BASIS + 1)[None, :]).astype(jnp.float32)
    bessel = jnp.sqrt(2.0 / CUTOFF) * jnp.sin(n * jnp.pi * u) / r
    ef = bessel * fc
    return rhat, ef


def _write_msg(msg_ref, rhat, ef, g, wrad_ref):
    r0 = ef @ wrad_ref[0]
    r1 = ef @ wrad_ref[1]
    r2 = ef @ wrad_ref[2]
    p0 = r0 * g
    p1 = r1 * g
    p2 = r2 * g
    comps = [p0]
    for d in range(3):
        comps.append(rhat[:, d:d + 1] * p1)
    for (i, j) in SYM_PAIRS:
        comps.append((rhat[:, i:i + 1] * rhat[:, j:j + 1]) * p2)
    msg_ref[...] = jnp.concatenate(comps, axis=1)


def _edge_body_l0(ev_ref, gat_ref, wrad_ref, msg_ref):
    rhat, ef = _edge_geom(ev_ref[...])
    g = gat_ref[:, 0:C]
    _write_msg(msg_ref, rhat, ef, g, wrad_ref)


def _edge_body_l1(ev_ref, gat_ref, wrad_ref, msg_ref):
    rhat, ef = _edge_geom(ev_ref[...])
    g = gat_ref[:, 0:C]
    for d in range(3):
        g = g + rhat[:, d:d + 1] * gat_ref[:, (1 + d) * C:(2 + d) * C]
    _write_msg(msg_ref, rhat, ef, g, wrad_ref)


EB = 1280


def _edge_msg(body, gat, edge_vector, wrad_l):
    grid = N_EDGES // EB
    d = gat.shape[1]
    return pl.pallas_call(
        body,
        grid=(grid,),
        in_specs=[
            pl.BlockSpec((3, EB), lambda i: (0, i)),
            pl.BlockSpec((EB, d), lambda i: (i, 0)),
            pl.BlockSpec((3, NUM_BASIS, C), lambda i: (0, 0, 0)),
        ],
        out_specs=pl.BlockSpec((EB, 10 * C), lambda i: (i, 0)),
        out_shape=jax.ShapeDtypeStruct((N_EDGES, 10 * C), jnp.float32),
    )(edge_vector, gat, wrad_l)


def _agg_comp(a, k):
    lo, hi = (k % 2) * C, (k % 2) * C + C
    if k < 8:
        return a[k // 2, :, lo:hi]
    return a[4, :, lo:hi] + a[5, :, lo:hi]


def _node_mix_inv(agg_ref, wmix_ref, wsq_ref):
    a = agg_ref[...] * (1.0 / AVG_NEIGH)
    a0 = _agg_comp(a, 0) @ wmix_ref[0]
    a1 = [_agg_comp(a, 1 + d) @ wmix_ref[1] for d in range(3)]
    a2 = [_agg_comp(a, 4 + k) @ wmix_ref[2] for k in range(6)]
    inv = (a0 * a0) @ wsq_ref[0]
    s1 = a1[0] * a1[0] + a1[1] * a1[1] + a1[2] * a1[2]
    inv = inv + s1 @ wsq_ref[1]
    s2 = SYM_W[0] * (a2[0] * a2[0])
    for k in range(1, 6):
        s2 = s2 + SYM_W[k] * (a2[k] * a2[k])
    inv = inv + s2 @ wsq_ref[2]
    return a0, a1, inv


def _node_body_l0(agg_ref, wmix_ref, wsq_ref, wout_ref, pk_ref):
    a0, a1, inv = _node_mix_inv(agg_ref, wmix_ref, wsq_ref)
    h0 = a0 @ wout_ref[0] + inv
    h1 = [a1[d] @ wout_ref[1] for d in range(3)]
    pk_ref[...] = jnp.concatenate([h0] + h1, axis=1)


def _node_body_l1(agg_ref, wmix_ref, wsq_ref, wout_ref, out_ref):
    a0, _, inv = _node_mix_inv(agg_ref, wmix_ref, wsq_ref)
    out_ref[...] = a0 @ wout_ref[0] + inv


def _node(body, out_minor, agg, wmix, wsq, wout):
    grid = N_NODES // NBK
    return pl.pallas_call(
        body,
        grid=(grid,),
        in_specs=[
            pl.BlockSpec((6, NBK, 2 * C), lambda i: (0, i, 0)),
            pl.BlockSpec((3, C, C), lambda i: (0, 0, 0)),
            pl.BlockSpec((3, C, C), lambda i: (0, 0, 0)),
            pl.BlockSpec((2, C, C), lambda i: (0, 0, 0)),
        ],
        out_specs=pl.BlockSpec((NBK, out_minor), lambda i: (i, 0)),
        out_shape=jax.ShapeDtypeStruct((N_NODES, out_minor), jnp.float32),
    )(agg, wmix, wsq, wout)


def kernel(node_attrs, edge_vector, edge_index, W_embed, Wrad, Wmix, Wsq, Wout):
    pad_n = NCHUNK_PAD * CH - N_EDGES
    src2 = jnp.pad(edge_index[0].astype(jnp.int32),
                   (0, pad_n)).reshape(NCHUNK_PAD, CH)
    dst = edge_index[1].astype(jnp.int32)
    evT = edge_vector.T
    n_sp = node_attrs.shape[1]
    attrs16 = jnp.pad(node_attrs, ((0, 0), (0, 16 - n_sp)))
    we16 = jnp.pad(W_embed, ((0, 16 - n_sp), (0, 0)))
    zeros_acc = jnp.zeros((N_NODES, 128), jnp.float32)

    table0 = _embed(attrs16, we16)
    gat0 = _sc_gather(table0, src2)
    msg0 = _edge_msg(_edge_body_l0, gat0, evT, Wrad[0])
    agg0 = _sc_scatter(msg0, dst, zeros_acc)
    packed = _node(_node_body_l0, 4 * C, agg0, Wmix[0], Wsq[0], Wout[0])

    gat1 = _sc_gather(packed, src2)
    msg1 = _edge_msg(_edge_body_l1, gat1, evT, Wrad[1])
    agg1 = _sc_scatter(msg1, dst, zeros_acc)
    return _node(_node_body_l1, C, agg1, Wmix[1], Wsq[1], Wout[1])


# speedup vs baseline: 51.6570x; 1.6868x over previous
import functools

import jax
import jax.numpy as jnp
from jax import lax
from jax.experimental import pallas as pl
from jax.experimental.pallas import tpu as pltpu
from jax.experimental.pallas import tpu_sc as plsc

N_NODES = 10000
N_EDGES = 160000
C = 64
NUM_BASIS = 8
CUTOFF = 6.0
AVG_NEIGH = 16.0

NC = 2
NS = 16
CH = 128
NCHUNK = N_EDGES // CH
NCHUNK_PAD = 1280
PER_G = NCHUNK_PAD // (NC * NS)
SPLITCHUNK = 624
LAG = 16

SYM_PAIRS = ((0, 0), (0, 1), (0, 2), (1, 1), (1, 2), (2, 2))
SYM_W = (1.0, 2.0, 2.0, 1.0, 2.0, 1.0)


def _gather_body_simple(table, idx_hbm, out, idx_v, rows_v, sem):
    wid = lax.axis_index("s") * NC + lax.axis_index("c")
    nw = NC * NS
    nloop = (NCHUNK + nw - 1) // nw

    def step(j, carry):
        cid = wid + nw * j

        @pl.when(cid < NCHUNK)
        def _():
            base = pl.multiple_of(cid * CH, CH)
            pltpu.sync_copy(idx_hbm.at[pl.ds(base, CH)], idx_v)
            pltpu.async_copy(table.at[idx_v], rows_v, sem).wait()
            pltpu.sync_copy(rows_v, out.at[pl.ds(base, CH)])

        return carry

    lax.fori_loop(0, nloop, step, 0)


def _gather_body(nbuf, table, idx2, out, ibuf,
                 r0, r1, r2, r3, sg0, sg1, sg2, sg3, so0, so1, so2, so3):
    rows = (r0, r1, r2, r3)
    sgs = (sg0, sg1, sg2, sg3)
    sos = (so0, so1, so2, so3)
    wid = lax.axis_index("s") * NC + lax.axis_index("c")
    c0 = pl.multiple_of(wid * PER_G, 8)
    pltpu.sync_copy(idx2.at[pl.ds(c0, PER_G)], ibuf)
    nouter = (PER_G + nbuf - 1) // nbuf

    def outer(g, carry):
        for b in range(nbuf):
            j = g * nbuf + b
            cid = c0 + j

            @pl.when((j < PER_G) & (cid < NCHUNK))
            def _(b=b, j=j, g=g):
                @pl.when(g > 0)
                def _():
                    pltpu.make_async_copy(
                        table.at[pl.ds(0, CH)], rows[b], sos[b]).wait()

                pltpu.async_copy(table.at[ibuf.at[j]], rows[b], sgs[b])

        for b in range(nbuf):
            j = g * nbuf + b
            cid = c0 + j

            @pl.when((j < PER_G) & (cid < NCHUNK))
            def _(b=b, j=j, cid=cid):
                pltpu.make_async_copy(
                    table.at[ibuf.at[j]], rows[b], sgs[b]).wait()
                base = pl.multiple_of(cid * CH, CH)
                pltpu.async_copy(rows[b], out.at[pl.ds(base, CH)], sos[b])

        return carry

    lax.fori_loop(0, nouter, outer, 0)
    for b in range(nbuf):
        @pl.when(c0 + b < NCHUNK)
        def _(b=b):
            pltpu.make_async_copy(
                table.at[pl.ds(0, CH)], rows[b], sos[b]).wait()


def _sc_gather(table, idx2):
    d = table.shape[1]
    nbuf = 4 if d <= 128 else 3
    mesh = plsc.VectorSubcoreMesh(core_axis_name="c", subcore_axis_name="s")
    f = pl.kernel(
        functools.partial(_gather_body, nbuf),
        out_type=jax.ShapeDtypeStruct((N_EDGES, d), jnp.float32),
        mesh=mesh,
        scratch_types=(
            [pltpu.VMEM((PER_G, CH), jnp.int32)]
            + [pltpu.VMEM((CH, d), jnp.float32)] * nbuf
            + [pltpu.VMEM((8, 128), jnp.float32)] * (4 - nbuf)
            + [pltpu.SemaphoreType.DMA] * 8
        ),
    )
    return f(table, idx2)


_SCAT_PLAN = (
    ((0, 0, 0, NCHUNK, 80), (1, 1, 0, NCHUNK, 80),
     (4, 4, 0, SPLITCHUNK, 40)),
    ((2, 2, 0, NCHUNK, 80), (3, 3, 0, NCHUNK, 80),
     (4, 5, SPLITCHUNK, NCHUNK, 40)),
)


_TROWS = 632
_TROWS_LAST = N_NODES - 15 * _TROWS


def _scatter_body(msg_hbm, dst_hbm, zeros_hbm, out_hbm,
                  i0, i1, m0, m1, acc, sl0, sl1, ss0, ss1):
    s = lax.axis_index("c")
    t = lax.axis_index("s")
    ib = (i0, i1)
    mb = (m0, m1)
    sls = (sl0, sl1)
    sss = (ss0, ss1)

    def each_tile_rows(fn):
        @pl.when(t < 15)
        def _():
            fn(pl.multiple_of(t * _TROWS, 8), _TROWS)

        @pl.when(t == 15)
        def _():
            fn(15 * _TROWS, _TROWS_LAST)

    def drain_load(b):
        pltpu.make_async_copy(
            dst_hbm.at[pl.ds(0, CH)], ib[b], sls[b]).wait()
        pltpu.make_async_copy(
            msg_hbm.at[pl.ds(0, CH), pl.ds(0, 128)], mb[b], sls[b]).wait()

    def drain_scat(b):
        pltpu.make_async_copy(mb[b], acc.at[ib[b]], sss[b]).wait()

    for sv in range(2):

        @pl.when(s == sv)
        def _(sv=sv):
            for (slot, slab, lo, hi, per) in _SCAT_PLAN[sv]:
                each_tile_rows(lambda r0, nr: pltpu.sync_copy(
                    zeros_hbm.at[pl.ds(r0, nr)], acc.at[pl.ds(r0, nr)]))
                plsc.subcore_barrier()
                c0 = pl.multiple_of(lo + t * per, 8)

                def body(g, carry, slot=slot, c0=c0, hi=hi):
                    for b in range(2):
                        j = 2 * g + b
                        cid = c0 + j

                        @pl.when(cid < hi)
                        def _(b=b, cid=cid, g=g):
                            @pl.when(g > 0)
                            def _():
                                drain_scat(b)

                            base = pl.multiple_of(cid * CH, CH)
                            pltpu.async_copy(
                                dst_hbm.at[pl.ds(base, CH)], ib[b], sls[b])
                            pltpu.async_copy(
                                msg_hbm.at[pl.ds(base, CH),
                                           pl.ds(slot * 128, 128)],
                                mb[b], sls[b])

                    for b in range(2):
                        j = 2 * g + b
                        cid = c0 + j

                        @pl.when(cid < hi)
                        def _(b=b):
                            drain_load(b)
                            pltpu.async_copy(mb[b], acc.at[ib[b]],
                                             sss[b], add=True)

                    return carry

                lax.fori_loop(0, per // 2, body, 0)
                for b in range(2):
                    @pl.when(c0 + b < hi)
                    def _(b=b):
                        drain_scat(b)

                plsc.subcore_barrier()
                each_tile_rows(lambda r0, nr, slab=slab: pltpu.sync_copy(
                    acc.at[pl.ds(r0, nr)],
                    out_hbm.at[slab, pl.ds(r0, nr)]))
                plsc.subcore_barrier()


def _sc_scatter(msg, dst_flat, zeros_acc):
    mesh = plsc.VectorSubcoreMesh(core_axis_name="c", subcore_axis_name="s")
    f = pl.kernel(
        _scatter_body,
        out_type=jax.ShapeDtypeStruct((6, N_NODES, 128), jnp.float32),
        mesh=mesh,
        scratch_types=(
            [pltpu.VMEM((CH,), jnp.int32)] * 2
            + [pltpu.VMEM((CH, 128), jnp.float32)] * 2
            + [pltpu.VMEM_SHARED((N_NODES, 128), jnp.float32)]
            + [pltpu.SemaphoreType.DMA] * 4
        ),
    )
    return f(msg, dst_flat, zeros_acc)


NBK = 2000


def _embed_body(at_ref, we_ref, out_ref):
    h = at_ref[...] @ we_ref[...]
    out_ref[...] = jnp.concatenate([h, jnp.zeros_like(h)], axis=1)


def _embed(attrs16, we16):
    grid = N_NODES // NBK
    return pl.pallas_call(
        _embed_body,
        grid=(grid,),
        in_specs=[
            pl.BlockSpec((NBK, 16), lambda i: (i, 0)),
            pl.BlockSpec((16, C), lambda i: (0, 0)),
        ],
        out_specs=pl.BlockSpec((NBK, 2 * C), lambda i: (i, 0)),
        out_shape=jax.ShapeDtypeStruct((N_NODES, 2 * C), jnp.float32),
    )(attrs16, we16)


def _edge_geom(evT):
    r2 = jnp.sum(evT * evT, axis=0, keepdims=True)
    r = jnp.sqrt(r2)
    inv_r = 1.0 / r
    rhat = evT * inv_r
    u = r * (1.0 / CUTOFF)
    u2 = u * u
    u4 = u2 * u2
    u5 = u4 * u
    u6 = u5 * u
    u7 = u6 * u
    fc = jnp.where(u < 1.0, 1.0 - 21.0 * u5 + 35.0 * u6 - 15.0 * u7, 0.0)
    n = (lax.broadcasted_iota(jnp.int32, (NUM_BASIS, 1), 0) + 1
         ).astype(jnp.float32)
    freq = n * (jnp.pi * u)
    bessel = jnp.sin(freq) * (jnp.sqrt(2.0 / CUTOFF) * inv_r)
    ef = bessel * fc
    ones = jnp.ones_like(r)
    sym = [rhat[i:i + 1] * rhat[j:j + 1] for (i, j) in SYM_PAIRS]
    ang10 = jnp.concatenate([ones, rhat] + sym, axis=0)
    return ang10, ef


def _dot_t(a, b):
    return lax.dot_general(a, b, (((0,), (0,)), ((), ())),
                           preferred_element_type=jnp.float32)


_RK_OF_COMP = (0, 1, 1, 1, 2, 2, 2, 2, 2, 2)


def _write_msg(msg_ref, cf, ef, g, wrad_ref):
    p = [_dot_t(ef, wrad_ref[rk]) * g for rk in range(3)]
    msg_ref[:, 0:C] = p[0]
    for k in range(1, 10):
        sel = slice(k * C, (k + 1) * C)
        msg_ref[:, sel] = cf[:, sel] * p[_RK_OF_COMP[k]]


def _edge_body_l0(ev_ref, gat_ref, wrad_ref, bsel_ref, msg_ref):
    ang10, ef = _edge_geom(ev_ref[...])
    cf = _dot_t(ang10, bsel_ref[...])
    g = gat_ref[:, 0:C]
    _write_msg(msg_ref, cf, ef, g, wrad_ref)


def _edge_body_l1(ev_ref, gat_ref, wrad_ref, bsel_ref, msg_ref):
    ang10, ef = _edge_geom(ev_ref[...])
    cf = _dot_t(ang10, bsel_ref[...])
    g = gat_ref[:, 0:C]
    for d in range(3):
        sel = slice((1 + d) * C, (2 + d) * C)
        g = g + cf[:, sel] * gat_ref[:, sel]
    _write_msg(msg_ref, cf, ef, g, wrad_ref)


EB = 1280


def _edge_msg(body, gat, edge_vector, wrad_l, bsel):
    grid = N_EDGES // EB
    d = gat.shape[1]
    return pl.pallas_call(
        body,
        grid=(grid,),
        in_specs=[
            pl.BlockSpec((3, EB), lambda i: (0, i)),
            pl.BlockSpec((EB, d), lambda i: (i, 0)),
            pl.BlockSpec((3, NUM_BASIS, C), lambda i: (0, 0, 0)),
            pl.BlockSpec((10, 10 * C), lambda i: (0, 0)),
        ],
        out_specs=pl.BlockSpec((EB, 10 * C), lambda i: (i, 0)),
        out_shape=jax.ShapeDtypeStruct((N_EDGES, 10 * C), jnp.float32),
    )(edge_vector, gat, wrad_l, bsel)


def _agg_comp(a, k):
    lo, hi = (k % 2) * C, (k % 2) * C + C
    if k < 8:
        return a[k // 2, :, lo:hi]
    return a[4, :, lo:hi] + a[5, :, lo:hi]


def _node_mix_inv(agg_ref, wmix_ref, wsq_ref):
    a = agg_ref[...] * (1.0 / AVG_NEIGH)
    a0 = _agg_comp(a, 0) @ wmix_ref[0]
    a1 = [_agg_comp(a, 1 + d) @ wmix_ref[1] for d in range(3)]
    a2 = [_agg_comp(a, 4 + k) @ wmix_ref[2] for k in range(6)]
    inv = (a0 * a0) @ wsq_ref[0]
    s1 = a1[0] * a1[0] + a1[1] * a1[1] + a1[2] * a1[2]
    inv = inv + s1 @ wsq_ref[1]
    s2 = SYM_W[0] * (a2[0] * a2[0])
    for k in range(1, 6):
        s2 = s2 + SYM_W[k] * (a2[k] * a2[k])
    inv = inv + s2 @ wsq_ref[2]
    return a0, a1, inv


def _node_body_l0(agg_ref, wmix_ref, wsq_ref, wout_ref, pk_ref):
    a0, a1, inv = _node_mix_inv(agg_ref, wmix_ref, wsq_ref)
    h0 = a0 @ wout_ref[0] + inv
    h1 = [a1[d] @ wout_ref[1] for d in range(3)]
    pk_ref[...] = jnp.concatenate([h0] + h1, axis=1)


def _node_body_l1(agg_ref, wmix_ref, wsq_ref, wout_ref, out_ref):
    a0, _, inv = _node_mix_inv(agg_ref, wmix_ref, wsq_ref)
    out_ref[...] = a0 @ wout_ref[0] + inv


def _node(body, out_minor, agg, wmix, wsq, wout):
    grid = N_NODES // NBK
    return pl.pallas_call(
        body,
        grid=(grid,),
        in_specs=[
            pl.BlockSpec((6, NBK, 2 * C), lambda i: (0, i, 0)),
            pl.BlockSpec((3, C, C), lambda i: (0, 0, 0)),
            pl.BlockSpec((3, C, C), lambda i: (0, 0, 0)),
            pl.BlockSpec((2, C, C), lambda i: (0, 0, 0)),
        ],
        out_specs=pl.BlockSpec((NBK, out_minor), lambda i: (i, 0)),
        out_shape=jax.ShapeDtypeStruct((N_NODES, out_minor), jnp.float32),
    )(agg, wmix, wsq, wout)


def kernel(node_attrs, edge_vector, edge_index, W_embed, Wrad, Wmix, Wsq, Wout):
    pad_n = NCHUNK_PAD * CH - N_EDGES
    src2 = jnp.pad(edge_index[0].astype(jnp.int32),
                   (0, pad_n)).reshape(NCHUNK_PAD, CH)
    dst = edge_index[1].astype(jnp.int32)
    evT = edge_vector.T
    n_sp = node_attrs.shape[1]
    attrs16 = jnp.pad(node_attrs, ((0, 0), (0, 16 - n_sp)))
    we16 = jnp.pad(W_embed, ((0, 16 - n_sp), (0, 0)))
    zeros_acc = jnp.zeros((N_NODES, 128), jnp.float32)
    bsel = jnp.kron(jnp.eye(10, dtype=jnp.float32),
                    jnp.ones((1, C), jnp.float32))

    table0 = _embed(attrs16, we16)
    gat0 = _sc_gather(table0, src2)
    msg0 = _edge_msg(_edge_body_l0, gat0, evT, Wrad[0], bsel)
    agg0 = _sc_scatter(msg0, dst, zeros_acc)
    packed = _node(_node_body_l0, 4 * C, agg0, Wmix[0], Wsq[0], Wout[0])

    gat1 = _sc_gather(packed, src2)
    msg1 = _edge_msg(_edge_body_l1, gat1, evT, Wrad[1], bsel)
    agg1 = _sc_scatter(msg1, dst, zeros_acc)
    return _node(_node_body_l1, C, agg1, Wmix[1], Wsq[1], Wout[1])
